# baseline (speedup 1.0000x reference)
import functools

import jax
import jax.numpy as jnp
from jax import lax
from jax.experimental import pallas as pl
from jax.experimental.pallas import tpu as pltpu
from jax.experimental.pallas import tpu_sc as plsc

_NC = 2
_NS = 16
_NW = _NC * _NS
_LANES = 16

_BATCH = 4
_SEQ = 8192
_HID = 2048
_CHUNK_ROWS = 8
_CHUNK = _CHUNK_ROWS * _HID
_ROWS_PER_W = _SEQ // _NW
_NCHUNK = _ROWS_PER_W // _CHUNK_ROWS
_UNROLL = 16


def _sc_body(x_hbm, pos_hbm, out_hbm,
             xbuf0, xbuf1, xbuf2, xbuf3, pbuf0, pbuf1,
             insem0, insem1, insem2, insem3,
             osem0, osem1, osem2, osem3, psem0, psem1):
    c_ax = lax.axis_index("c")
    s_ax = lax.axis_index("s")
    wid = s_ax * _NC + c_ax
    seq0 = wid * _ROWS_PER_W

    xbufs = (xbuf0, xbuf1, xbuf2, xbuf3)
    pbufs = (pbuf0, pbuf1)
    insems = (insem0, insem1, insem2, insem3)
    osems = (osem0, osem1, osem2, osem3)
    psems = (psem0, psem1)

    def pos_off(c):
        return (seq0 + c * _CHUNK_ROWS) * _HID

    def x_off(c, b):
        return (b * _SEQ + seq0 + c * _CHUNK_ROWS) * _HID

    def start_in(c, b, p):
        pltpu.async_copy(x_hbm.at[pl.ds(x_off(c, b), _CHUNK)], xbufs[p],
                         insems[p])

    def start_pos(c, p):
        pltpu.async_copy(pos_hbm.at[pl.ds(pos_off(c), _CHUNK)], pbufs[p],
                         psems[p])

    def start_out(c, b, p):
        pltpu.async_copy(xbufs[p], out_hbm.at[pl.ds(x_off(c, b), _CHUNK)],
                         osems[p])

    def wait_in(p):
        pltpu.make_async_copy(x_hbm.at[pl.ds(0, _CHUNK)], xbufs[p],
                              insems[p]).wait()

    def wait_pos(p):
        pltpu.make_async_copy(pos_hbm.at[pl.ds(0, _CHUNK)], pbufs[p],
                              psems[p]).wait()

    def wait_out(p):
        pltpu.make_async_copy(xbufs[p], out_hbm.at[pl.ds(0, _CHUNK)],
                              osems[p]).wait()

    def compute(xp, pp):
        xb = xbufs[xp]
        pb = pbufs[pp]

        @plsc.parallel_loop(0, _CHUNK, step=_LANES, unroll=_UNROLL)
        def _(i):
            sl = pl.ds(i, _LANES)
            plsc.addupdate(xb.at[sl], pb[sl])

    start_pos(0, 0)
    start_in(0, 0, 0)
    start_in(0, 1, 1)
    start_in(0, 2, 2)

    def chunk_pair(half, carry):
        cc = half * 2
        for cu in range(2):
            c = cc + cu
            for b in range(4):
                nxt = (b + 3) % 4
                if b == 0:

                    @pl.when(c > 0)
                    def _():
                        wait_out(nxt)

                    start_in(c, 3, nxt)

                    @pl.when(c + 1 < _NCHUNK)
                    def _():
                        start_pos(c + 1, 1 - cu)

                    wait_pos(cu)
                else:
                    wait_out(nxt)

                    @pl.when(c + 1 < _NCHUNK)
                    def _():
                        start_in(c + 1, b - 1, nxt)

                wait_in(b)
                compute(b, cu)
                start_out(c, b, b)
        return carry

    lax.fori_loop(0, _NCHUNK // 2, chunk_pair, 0)
    wait_out(3)


def kernel(x, pos_embedding):
    batch, seq_len, hidden = x.shape
    pos = pos_embedding[:seq_len]
    x_flat = x.reshape(-1)
    pos_flat = pos.reshape(-1)

    mesh = plsc.VectorSubcoreMesh(core_axis_name="c", subcore_axis_name="s")
    call = functools.partial(
        pl.kernel,
        out_type=jax.ShapeDtypeStruct((batch * seq_len * hidden,), x.dtype),
        mesh=mesh,
        scratch_types=(
            [pltpu.VMEM((_CHUNK,), jnp.float32)] * 6
            + [pltpu.SemaphoreType.DMA] * 10
        ),
    )(_sc_body)
    out_flat = call(x_flat, pos_flat)
    return out_flat.reshape(x.shape)


# speedup vs baseline: 2.8688x; 2.8688x over previous
import functools

import jax
import jax.numpy as jnp
from jax import lax
from jax.experimental import pallas as pl
from jax.experimental.pallas import tpu as pltpu
from jax.experimental.pallas import tpu_sc as plsc

_NC = 2
_NS = 16
_NW = _NC * _NS
_LANES = 16

_BATCH = 4
_SEQ = 8192
_HID = 2048
_CHUNK_ROWS = 8
_ROWS_PER_W = _SEQ // _NW
_NCHUNK = _ROWS_PER_W // _CHUNK_ROWS
_UNROLL = 8


def _sc_body(x_hbm, pos_hbm, out_hbm,
             xbuf0, xbuf1, xbuf2, xbuf3, pbuf0, pbuf1,
             insem0, insem1, insem2, insem3,
             osem0, osem1, osem2, osem3, psem0, psem1):
    c_ax = lax.axis_index("c")
    s_ax = lax.axis_index("s")
    wid = s_ax * _NC + c_ax
    seq0 = wid * _ROWS_PER_W

    xbufs = (xbuf0, xbuf1, xbuf2, xbuf3)
    pbufs = (pbuf0, pbuf1)
    insems = (insem0, insem1, insem2, insem3)
    osems = (osem0, osem1, osem2, osem3)
    psems = (psem0, psem1)

    def pos_row(c):
        return seq0 + c * _CHUNK_ROWS

    def x_row(c, b):
        return b * _SEQ + seq0 + c * _CHUNK_ROWS

    def start_in(c, b, p):
        pltpu.async_copy(x_hbm.at[pl.ds(x_row(c, b), _CHUNK_ROWS)], xbufs[p],
                         insems[p])

    def start_pos(c, p):
        pltpu.async_copy(pos_hbm.at[pl.ds(pos_row(c), _CHUNK_ROWS)], pbufs[p],
                         psems[p])

    def start_out(c, b, p):
        pltpu.async_copy(xbufs[p], out_hbm.at[pl.ds(x_row(c, b), _CHUNK_ROWS)],
                         osems[p])

    def wait_in(p):
        pltpu.make_async_copy(x_hbm.at[pl.ds(0, _CHUNK_ROWS)], xbufs[p],
                              insems[p]).wait()

    def wait_pos(p):
        pltpu.make_async_copy(pos_hbm.at[pl.ds(0, _CHUNK_ROWS)], pbufs[p],
                              psems[p]).wait()

    def wait_out(p):
        pltpu.make_async_copy(xbufs[p], out_hbm.at[pl.ds(0, _CHUNK_ROWS)],
                              osems[p]).wait()

    def compute(xp, pp):
        xb = xbufs[xp]
        pb = pbufs[pp]
        for r in range(_CHUNK_ROWS):

            @plsc.parallel_loop(0, _HID, step=_LANES, unroll=_UNROLL)
            def _(i):
                sl = pl.ds(i, _LANES)
                plsc.addupdate(xb.at[r, sl], pb[r, sl])

    start_pos(0, 0)
    start_in(0, 0, 0)
    start_in(0, 1, 1)
    start_in(0, 2, 2)

    def chunk_pair(half, carry):
        cc = half * 2
        for cu in range(2):
            c = cc + cu
            for b in range(4):
                nxt = (b + 3) % 4
                if b == 0:

                    @pl.when(c > 0)
                    def _():
                        wait_out(nxt)

                    start_in(c, 3, nxt)

                    @pl.when(c + 1 < _NCHUNK)
                    def _():
                        start_pos(c + 1, 1 - cu)

                    wait_pos(cu)
                else:
                    wait_out(nxt)

                    @pl.when(c + 1 < _NCHUNK)
                    def _():
                        start_in(c + 1, b - 1, nxt)

                wait_in(b)
                compute(b, cu)
                start_out(c, b, b)
        return carry

    lax.fori_loop(0, _NCHUNK // 2, chunk_pair, 0)
    wait_out(3)


def kernel(x, pos_embedding):
    batch, seq_len, hidden = x.shape
    pos = pos_embedding[:seq_len]
    x2 = x.reshape(batch * seq_len, hidden)

    mesh = plsc.VectorSubcoreMesh(core_axis_name="c", subcore_axis_name="s")
    call = functools.partial(
        pl.kernel,
        out_type=jax.ShapeDtypeStruct((batch * seq_len, hidden), x.dtype),
        mesh=mesh,
        scratch_types=(
            [pltpu.VMEM((_CHUNK_ROWS, _HID), jnp.float32)] * 6
            + [pltpu.SemaphoreType.DMA] * 10
        ),
    )(_sc_body)
    out2 = call(x2, pos)
    return out2.reshape(x.shape)
